# baseline (speedup 1.0000x reference)
import functools

import jax
import jax.numpy as jnp
from jax import lax
from jax.experimental import pallas as pl
from jax.experimental.pallas import tpu as pltpu
from jax.experimental.pallas import tpu_sc as plsc

BATCH = 16384
FACTORS = 64

_info = plsc.get_sparse_core_info()
NC = _info.num_cores
NS = _info.num_subcores
NW = NC * NS
B_PER_W = BATCH // NW
IDX_CHUNK = 128
N_CHUNKS = B_PER_W // IDX_CHUNK
N_BUF = 2


def _pointmf_kernel(uh_hbm, uo_hbm, ih_hbm, io_hbm, eu_hbm, ei_hbm, out_hbm,
                    uh_v, uo_v, ih_v, io_v, eu_b, ei_b, out_v, sem_u, sem_i):
    wid = lax.axis_index("s") * NC + lax.axis_index("c")
    base = wid * B_PER_W

    pltpu.sync_copy(uh_hbm.at[pl.ds(base, B_PER_W)], uh_v)
    pltpu.sync_copy(uo_hbm.at[pl.ds(base, B_PER_W)], uo_v)
    pltpu.sync_copy(ih_hbm.at[pl.ds(base, B_PER_W)], ih_v)
    pltpu.sync_copy(io_hbm.at[pl.ds(base, B_PER_W)], io_v)

    def fire(j):
        idx = pl.ds(j * IDX_CHUNK, IDX_CHUNK)
        slot = j % N_BUF
        cu = pltpu.async_copy(eu_hbm.at[uh_v.at[idx]], eu_b.at[slot],
                              sem_u.at[slot])
        ci = pltpu.async_copy(ei_hbm.at[ih_v.at[idx]], ei_b.at[slot],
                              sem_i.at[slot])
        return cu, ci

    lanes = lax.iota(jnp.int32, 16)
    inflight = [fire(0), fire(1)]

    for j in range(N_CHUNKS):
        cu, ci = inflight[j % N_BUF]
        cu.wait()
        ci.wait()
        slot = j % N_BUF
        eu_s = eu_b.at[slot]
        ei_s = ei_b.at[slot]

        def group(g, _):
            rows = g * 16 + lanes
            offu = uo_v[pl.ds(j * IDX_CHUNK + g * 16, 16)]
            offi = io_v[pl.ds(j * IDX_CHUNK + g * 16, 16)]

            def fstep(f, acc):
                cols = (lanes + f) & 63
                a = plsc.load_gather(eu_s, [rows, cols + offu])
                b = plsc.load_gather(ei_s, [rows, cols + offi])
                return acc + a * b

            acc = lax.fori_loop(0, FACTORS, fstep,
                                jnp.zeros((16,), jnp.float32), unroll=8)
            out_v[pl.ds(j * IDX_CHUNK + g * 16, 16)] = acc
            return 0

        lax.fori_loop(0, IDX_CHUNK // 16, group, 0)
        if j + N_BUF < N_CHUNKS:
            inflight[j % N_BUF] = fire(j + N_BUF)

    pltpu.sync_copy(out_v, out_hbm.at[pl.ds(base, B_PER_W)])


@jax.jit
def _run(uh, uo, ih, io, eu2, ei2):
    mesh = plsc.VectorSubcoreMesh(core_axis_name="c", subcore_axis_name="s")
    f = functools.partial(
        pl.kernel,
        mesh=mesh,
        compiler_params=pltpu.CompilerParams(
            needs_layout_passes=False, use_tc_tiling_on_sc=True),
        out_type=jax.ShapeDtypeStruct((BATCH,), jnp.float32),
        scratch_types=[
            pltpu.VMEM((B_PER_W,), jnp.int32),
            pltpu.VMEM((B_PER_W,), jnp.int32),
            pltpu.VMEM((B_PER_W,), jnp.int32),
            pltpu.VMEM((B_PER_W,), jnp.int32),
            pltpu.VMEM((N_BUF, IDX_CHUNK, 2 * FACTORS), jnp.float32),
            pltpu.VMEM((N_BUF, IDX_CHUNK, 2 * FACTORS), jnp.float32),
            pltpu.VMEM((B_PER_W,), jnp.float32),
            pltpu.SemaphoreType.DMA((N_BUF,)),
            pltpu.SemaphoreType.DMA((N_BUF,)),
        ],
    )(_pointmf_kernel)
    return f(uh, uo, ih, io, eu2, ei2)


def kernel(user, item, embed_user, embed_item):
    u = user.astype(jnp.int32)
    it = item.astype(jnp.int32)
    uh = u >> 1
    uo = (u & 1) << 6
    ih = it >> 1
    io = (it & 1) << 6
    eu2 = embed_user.reshape(500000, 2 * FACTORS)
    ei2 = embed_item.reshape(500000, 2 * FACTORS)
    return _run(uh, uo, ih, io, eu2, ei2)
